# baseline (speedup 1.0000x reference)
import functools
import math

import jax
import jax.numpy as jnp
from jax import lax
from jax.experimental import pallas as pl
from jax.experimental.pallas import tpu as pltpu
from jax.experimental.pallas import tpu_sc as plsc

D_MODEL = 1024
SCALE = math.sqrt(D_MODEL)

NC = 2
NS = 16
LANES = 16
NW = NC * NS

CHUNK = 8
NBUF = 8
SLACK = 3
LOOK = NBUF - SLACK
VECS_PER_ROW = D_MODEL // LANES


@functools.cache
def _build(B):
  n_per_w = B // NW
  n_chunks = n_per_w // CHUNK
  assert n_chunks % NBUF == 0
  n_turns = n_chunks // NBUF

  mesh = plsc.VectorSubcoreMesh(core_axis_name="c", subcore_axis_name="s")

  @functools.partial(
      pl.kernel,
      out_type=jax.ShapeDtypeStruct((B, D_MODEL), jnp.float32),
      mesh=mesh,
      scratch_types=[
          pltpu.VMEM((n_per_w,), jnp.int32),
      ] + [pltpu.VMEM((CHUNK, D_MODEL), jnp.float32)] * NBUF
        + [pltpu.SemaphoreType.DMA] * (2 * NBUF),
  )
  def emb_kernel(idx_hbm, table_hbm, out_hbm, idx_v, *bufs_sems):
    bufs = bufs_sems[:NBUF]
    gsems = bufs_sems[NBUF:2 * NBUF]
    osems = bufs_sems[2 * NBUF:]

    wid = lax.axis_index("s") * NC + lax.axis_index("c")
    base = wid * n_per_w

    seq_len = idx_hbm.shape[1]
    pltpu.sync_copy(
        idx_hbm.at[(wid * n_per_w) // seq_len,
                   pl.ds((wid * n_per_w) % seq_len, n_per_w)], idx_v)

    def start_gather(c, b):
      pltpu.async_copy(
          table_hbm.at[idx_v.at[pl.ds(c * CHUNK, CHUNK)]], bufs[b], gsems[b])

    def wait_gather(b):
      pltpu.make_async_copy(table_hbm.at[idx_v.at[pl.ds(0, CHUNK)]], bufs[b],
                            gsems[b]).wait()

    def start_scatter(c, b):
      pltpu.async_copy(
          bufs[b], out_hbm.at[pl.ds(base + c * CHUNK, CHUNK)], osems[b])

    def wait_scatter(b):
      pltpu.make_async_copy(bufs[b], out_hbm.at[pl.ds(base, CHUNK)],
                            osems[b]).wait()

    for b in range(LOOK):
      start_gather(b, b)

    def turn(g, carry):
      for b in range(NBUF):
        c = g * NBUF + b
        wait_gather(b)

        def scale_row(r, acc, buf=bufs[b]):
          for j in range(VECS_PER_ROW):
            buf[r, pl.ds(j * LANES, LANES)] = (
                buf[r, pl.ds(j * LANES, LANES)] * SCALE)
          return acc

        lax.fori_loop(0, CHUNK, scale_row, 0, unroll=False)

        start_scatter(c, b)

        nxt = c + LOOK
        nb = (b + LOOK) % NBUF

        @pl.when(nxt < n_chunks)
        def _():
          @pl.when(c >= SLACK)
          def _():
            wait_scatter(nb)
          start_gather(nxt, nb)
      return carry

    lax.fori_loop(0, n_turns, turn, 0, unroll=False)

    for b in range(NBUF):
      wait_scatter(b)

  return emb_kernel


def kernel(x, table):
  orig_shape = x.shape
  B = x.size
  idx = x.reshape(orig_shape[0], -1).astype(jnp.int32)
  out = _build(B)(idx, table)
  return out.reshape(*orig_shape, D_MODEL)


# speedup vs baseline: 1.0140x; 1.0140x over previous
import functools
import math

import jax
import jax.numpy as jnp
from jax import lax
from jax.experimental import pallas as pl
from jax.experimental.pallas import tpu as pltpu
from jax.experimental.pallas import tpu_sc as plsc

D_MODEL = 1024
SCALE = math.sqrt(D_MODEL)

NC = 2
NS = 16
LANES = 16
NW = NC * NS

CHUNK = 8
NBUF = 8
SLACK = 1
LOOK = NBUF - SLACK
VECS_PER_ROW = D_MODEL // LANES


@functools.cache
def _build(B):
  n_per_w = B // NW
  n_chunks = n_per_w // CHUNK
  assert n_chunks % NBUF == 0
  n_turns = n_chunks // NBUF

  mesh = plsc.VectorSubcoreMesh(core_axis_name="c", subcore_axis_name="s")

  @functools.partial(
      pl.kernel,
      out_type=jax.ShapeDtypeStruct((B, D_MODEL), jnp.float32),
      mesh=mesh,
      scratch_types=[
          pltpu.VMEM((n_per_w,), jnp.int32),
      ] + [pltpu.VMEM((CHUNK, D_MODEL), jnp.float32)] * NBUF
        + [pltpu.SemaphoreType.DMA] * (2 * NBUF),
  )
  def emb_kernel(idx_hbm, table_hbm, out_hbm, idx_v, *bufs_sems):
    bufs = bufs_sems[:NBUF]
    gsems = bufs_sems[NBUF:2 * NBUF]
    osems = bufs_sems[2 * NBUF:]

    wid = lax.axis_index("s") * NC + lax.axis_index("c")
    base = wid * n_per_w

    seq_len = idx_hbm.shape[1]
    pltpu.sync_copy(
        idx_hbm.at[(wid * n_per_w) // seq_len,
                   pl.ds((wid * n_per_w) % seq_len, n_per_w)], idx_v)

    def start_gather(c, b):
      pltpu.async_copy(
          table_hbm.at[idx_v.at[pl.ds(c * CHUNK, CHUNK)]], bufs[b], gsems[b])

    def wait_gather(b):
      pltpu.make_async_copy(table_hbm.at[idx_v.at[pl.ds(0, CHUNK)]], bufs[b],
                            gsems[b]).wait()

    def start_scatter(c, b):
      pltpu.async_copy(
          bufs[b], out_hbm.at[pl.ds(base + c * CHUNK, CHUNK)], osems[b])

    def wait_scatter(b):
      pltpu.make_async_copy(bufs[b], out_hbm.at[pl.ds(base, CHUNK)],
                            osems[b]).wait()

    for b in range(LOOK):
      start_gather(b, b)

    def turn(g, carry):
      for b in range(NBUF):
        c = g * NBUF + b
        wait_gather(b)

        def scale_row(r, acc, buf=bufs[b]):
          for j in range(VECS_PER_ROW):
            buf[r, pl.ds(j * LANES, LANES)] = (
                buf[r, pl.ds(j * LANES, LANES)] * SCALE)
          return acc

        lax.fori_loop(0, CHUNK, scale_row, 0, unroll=False)

        start_scatter(c, b)

        nxt = c + LOOK
        nb = (b + LOOK) % NBUF

        @pl.when(nxt < n_chunks)
        def _():
          @pl.when(c >= SLACK)
          def _():
            wait_scatter(nb)
          start_gather(nxt, nb)
      return carry

    lax.fori_loop(0, n_turns, turn, 0, unroll=False)

    for b in range(NBUF):
      wait_scatter(b)

  return emb_kernel


def kernel(x, table):
  orig_shape = x.shape
  B = x.size
  idx = x.reshape(orig_shape[0], -1).astype(jnp.int32)
  out = _build(B)(idx, table)
  return out.reshape(*orig_shape, D_MODEL)
